# Initial kernel scaffold; baseline (speedup 1.0000x reference)
import jax
import jax.numpy as jnp
from jax.experimental import pallas as pl


def kernel(x, W_rels, W_loops, b_rels, gat_W, gat_al, gat_ar, gat_b, dense_W, dense_b, edge_index, edge_type):
    raise NotImplementedError("write your pallas kernel here")


# speedup vs baseline: 34.8098x; 34.8098x over previous
import functools

import jax
import jax.numpy as jnp
from jax import lax
from jax.experimental import pallas as pl
from jax.experimental.pallas import tpu as pltpu
from jax.experimental.pallas import tpu_sc as plsc

N = 10000
D = 128
NRELS = 4
NLAYERS = 4
E = 320000

NP = 10240
EP = 327680
EW = EP // 32
CHUNK = 128
NCHUNKS = EW // CHUNK
ROWS_PER_SUB = NP // 16
BN = 1024
GRID = NP // BN

_mesh = functools.partial(
    plsc.VectorSubcoreMesh, core_axis_name="c", subcore_axis_name="s",
    num_cores=2, num_subcores=16)

_SC_PARAMS = pltpu.CompilerParams(needs_layout_passes=False)

f32 = jnp.float32
i32 = jnp.int32


def _conv_edges_body(hr, srcp, typep, dstp, out,
                     rows2, sidx, tidx, didx, gidx, acc, sem0, sem1):
  c = lax.axis_index("c")
  s = lax.axis_index("s")
  wid = c * 16 + s
  base = wid * EW
  sems = [sem0, sem1]

  @pl.loop(0, 128)
  def _(i):
    for j in range(D // 16):
      rows2[0, i, pl.ds(j * 16, 16)] = jnp.zeros((16,), f32)

  for k in range(ROWS_PER_SUB // 128):
    pltpu.sync_copy(rows2.at[0], acc.at[pl.ds(s * ROWS_PER_SUB + k * 128, 128)])
  plsc.subcore_barrier()

  def load_chunk(g, b):
    off = base + g * CHUNK
    pltpu.sync_copy(srcp.at[pl.ds(off, CHUNK)], sidx.at[b])
    pltpu.sync_copy(typep.at[pl.ds(off, CHUNK)], tidx.at[b])
    pltpu.sync_copy(dstp.at[pl.ds(off, CHUNK)], didx.at[b])
    for j in range(CHUNK // 16):
      sl = pl.ds(j * 16, 16)
      gidx[b, sl] = tidx[b, sl] * NP + sidx[b, sl]
    pltpu.async_copy(hr.at[gidx.at[b]], rows2.at[b], sems[b])

  load_chunk(0, 0)
  load_chunk(1, 1)

  @pl.loop(0, NCHUNKS // 2)
  def _(it):
    for b in range(2):
      g = it * 2 + b
      pltpu.make_async_copy(hr.at[gidx.at[b]], rows2.at[b], sems[b]).wait()
      pltpu.sync_copy(rows2.at[b], acc.at[didx.at[b]], add=True)

      @pl.when(g + 2 < NCHUNKS)
      def _():
        load_chunk(g + 2, b)

  plsc.subcore_barrier()
  pltpu.sync_copy(acc.at[pl.ds(s * ROWS_PER_SUB, ROWS_PER_SUB)],
                  out.at[c, pl.ds(s * ROWS_PER_SUB, ROWS_PER_SUB)])


def _conv_edges(hr_flat, srcp, typep, dstp):
  return pl.kernel(
      _conv_edges_body,
      out_type=jax.ShapeDtypeStruct((2, NP, D), f32),
      mesh=_mesh(),
      compiler_params=_SC_PARAMS,
      scratch_types=[
          pltpu.VMEM((2, CHUNK, D), f32),
          pltpu.VMEM((2, CHUNK), i32),
          pltpu.VMEM((2, CHUNK), i32),
          pltpu.VMEM((2, CHUNK), i32),
          pltpu.VMEM((2, CHUNK), i32),
          pltpu.VMEM_SHARED((NP, D), f32),
          pltpu.SemaphoreType.DMA,
          pltpu.SemaphoreType.DMA,
      ],
  )(hr_flat, srcp, typep, dstp)


def _gat_max_body(eo8, srcp, dstp, out, el_v, er_v, m_v, sbuf, dbuf,
                  redv, tmpv, msh):
  c = lax.axis_index("c")
  s = lax.axis_index("s")
  wid = c * 16 + s
  base = wid * EW

  pltpu.sync_copy(eo8.at[0], el_v)
  pltpu.sync_copy(eo8.at[1], er_v)

  @pl.loop(0, NP // 16)
  def _(i):
    m_v[pl.ds(i * 16, 16)] = jnp.full((16,), -jnp.inf, f32)

  @pl.loop(0, EW // 512)
  def _(k):
    off = base + k * 512
    pltpu.sync_copy(srcp.at[pl.ds(off, 512)], sbuf)
    pltpu.sync_copy(dstp.at[pl.ds(off, 512)], dbuf)
    for j in range(32):
      sl = pl.ds(j * 16, 16)
      s16 = sbuf[sl]
      d16 = dbuf[sl]
      x = plsc.load_gather(el_v, [s16]) + plsc.load_gather(er_v, [d16])
      e16 = jnp.where(x > 0, x, 0.2 * x)
      cur = plsc.load_gather(m_v, [d16])

      def _again(cur):
        return jnp.max(jnp.where(e16 > cur, 1, 0)) > 0

      def _push(cur):
        plsc.store_scatter(m_v, [d16], jnp.maximum(e16, cur), mask=e16 > cur)
        return plsc.load_gather(m_v, [d16])

      lax.while_loop(_again, _push, cur)

  pltpu.sync_copy(m_v, msh.at[s])
  plsc.subcore_barrier()
  colbase = s * ROWS_PER_SUB
  pltpu.sync_copy(msh.at[0, pl.ds(colbase, ROWS_PER_SUB)], redv)
  for k in range(1, 16):
    pltpu.sync_copy(msh.at[k, pl.ds(colbase, ROWS_PER_SUB)], tmpv)
    for i in range(ROWS_PER_SUB // 16):
      sl = pl.ds(i * 16, 16)
      redv[sl] = jnp.maximum(redv[sl], tmpv[sl])
  pltpu.sync_copy(redv, out.at[c, pl.ds(colbase, ROWS_PER_SUB)])


def _gat_max(eo8, srcp, dstp):
  return pl.kernel(
      _gat_max_body,
      out_type=jax.ShapeDtypeStruct((2, NP), f32),
      mesh=_mesh(),
      compiler_params=_SC_PARAMS,
      scratch_types=[
          pltpu.VMEM((NP,), f32),
          pltpu.VMEM((NP,), f32),
          pltpu.VMEM((NP,), f32),
          pltpu.VMEM((512,), i32),
          pltpu.VMEM((512,), i32),
          pltpu.VMEM((ROWS_PER_SUB,), f32),
          pltpu.VMEM((ROWS_PER_SUB,), f32),
          pltpu.VMEM_SHARED((16, NP), f32),
      ],
  )(eo8, srcp, dstp)


def _gat_sums_body(eo8, srcp, dstp, m_parts, den_out, num_out,
                   el_v, er_v, p_v, m_v, mtmp, sbuf, dbuf,
                   eebuf, epbuf, didx2, redv, den_sp, num_sp):
  c = lax.axis_index("c")
  s = lax.axis_index("s")
  wid = c * 16 + s
  base = wid * EW

  pltpu.sync_copy(eo8.at[0], el_v)
  pltpu.sync_copy(eo8.at[1], er_v)
  pltpu.sync_copy(eo8.at[2], p_v)
  pltpu.sync_copy(m_parts.at[0], m_v)
  pltpu.sync_copy(m_parts.at[1], mtmp)

  @pl.loop(0, NP // 16)
  def _(i):
    sl = pl.ds(i * 16, 16)
    m_v[sl] = jnp.maximum(m_v[sl], mtmp[sl])

  @pl.loop(0, ROWS_PER_SUB // 16)
  def _(i):
    redv[pl.ds(i * 16, 16)] = jnp.zeros((16,), f32)
  colbase = s * ROWS_PER_SUB
  pltpu.sync_copy(redv, den_sp.at[pl.ds(colbase, ROWS_PER_SUB)])
  pltpu.sync_copy(redv, num_sp.at[pl.ds(colbase, ROWS_PER_SUB)])
  plsc.subcore_barrier()

  @pl.loop(0, EW // 512)
  def _(k):
    off = base + k * 512
    pltpu.sync_copy(srcp.at[pl.ds(off, 512)], sbuf)
    pltpu.sync_copy(dstp.at[pl.ds(off, 512)], dbuf)
    for j in range(32):
      sl = pl.ds(j * 16, 16)
      s16 = sbuf[sl]
      d16 = dbuf[sl]
      x = plsc.load_gather(el_v, [s16]) + plsc.load_gather(er_v, [d16])
      e16 = jnp.where(x > 0, x, 0.2 * x)
      ee = jnp.exp(e16 - plsc.load_gather(m_v, [d16]))
      pg = plsc.load_gather(p_v, [s16])
      r, col = j // 8, pl.ds((j % 8) * 16, 16)
      eebuf[r, col] = ee
      epbuf[r, col] = ee * pg
      didx2[r, col] = d16
    for r in range(4):
      pltpu.sync_copy(eebuf.at[r], den_sp.at[didx2.at[r]], add=True)
      pltpu.sync_copy(epbuf.at[r], num_sp.at[didx2.at[r]], add=True)

  plsc.subcore_barrier()
  pltpu.sync_copy(den_sp.at[pl.ds(colbase, ROWS_PER_SUB)], redv)
  pltpu.sync_copy(redv, den_out.at[c, pl.ds(colbase, ROWS_PER_SUB)])
  pltpu.sync_copy(num_sp.at[pl.ds(colbase, ROWS_PER_SUB)], redv)
  pltpu.sync_copy(redv, num_out.at[c, pl.ds(colbase, ROWS_PER_SUB)])


def _gat_sums(eo8, srcp, dstp, m_parts):
  return pl.kernel(
      _gat_sums_body,
      out_type=[jax.ShapeDtypeStruct((2, NP), f32),
                jax.ShapeDtypeStruct((2, NP), f32)],
      mesh=_mesh(),
      compiler_params=_SC_PARAMS,
      scratch_types=[
          pltpu.VMEM((NP,), f32),
          pltpu.VMEM((NP,), f32),
          pltpu.VMEM((NP,), f32),
          pltpu.VMEM((NP,), f32),
          pltpu.VMEM((NP,), f32),
          pltpu.VMEM((512,), i32),
          pltpu.VMEM((512,), i32),
          pltpu.VMEM((4, 128), f32),
          pltpu.VMEM((4, 128), f32),
          pltpu.VMEM((4, 128), i32),
          pltpu.VMEM((ROWS_PER_SUB,), f32),
          pltpu.VMEM_SHARED((NP,), f32),
          pltpu.VMEM_SHARED((NP,), f32),
      ],
  )(eo8, srcp, dstp, m_parts)


def _mm_step_body(relu_in, h_ref, wr_ref, wl_ref, b_ref, hr_ref, selfb_ref,
                  p0_ref=None):
  if relu_in:
    h = jnp.maximum(p0_ref[0] + p0_ref[1] + h_ref[...], 0.0)
  else:
    h = h_ref[...]
  for r in range(NRELS):
    hr_ref[r] = jnp.dot(h, wr_ref[r], preferred_element_type=f32)
  selfb_ref[...] = jnp.dot(h, wl_ref[...], preferred_element_type=f32) + b_ref[...]


def _tc_first(xp, wr, wl, b2):
  body = functools.partial(_mm_step_body, False)

  def wrapped(h_ref, wr_ref, wl_ref, b_ref, hr_ref, selfb_ref):
    body(h_ref, wr_ref, wl_ref, b_ref, hr_ref, selfb_ref)

  return pl.pallas_call(
      wrapped,
      grid=(GRID,),
      in_specs=[
          pl.BlockSpec((BN, D), lambda i: (i, 0)),
          pl.BlockSpec((NRELS, D, D), lambda i: (0, 0, 0)),
          pl.BlockSpec((D, D), lambda i: (0, 0)),
          pl.BlockSpec((1, D), lambda i: (0, 0)),
      ],
      out_specs=[
          pl.BlockSpec((NRELS, BN, D), lambda i: (0, i, 0)),
          pl.BlockSpec((BN, D), lambda i: (i, 0)),
      ],
      out_shape=[jax.ShapeDtypeStruct((NRELS, NP, D), f32),
                 jax.ShapeDtypeStruct((NP, D), f32)],
  )(xp, wr, wl, b2)


def _tc_mid(parts, selfb, wr, wl, b2):
  def wrapped(p0_ref, h_ref, wr_ref, wl_ref, b_ref, hr_ref, selfb_ref):
    _mm_step_body(True, h_ref, wr_ref, wl_ref, b_ref, hr_ref, selfb_ref,
                  p0_ref=p0_ref)

  return pl.pallas_call(
      wrapped,
      grid=(GRID,),
      in_specs=[
          pl.BlockSpec((2, BN, D), lambda i: (0, i, 0)),
          pl.BlockSpec((BN, D), lambda i: (i, 0)),
          pl.BlockSpec((NRELS, D, D), lambda i: (0, 0, 0)),
          pl.BlockSpec((D, D), lambda i: (0, 0)),
          pl.BlockSpec((1, D), lambda i: (0, 0)),
      ],
      out_specs=[
          pl.BlockSpec((NRELS, BN, D), lambda i: (0, i, 0)),
          pl.BlockSpec((BN, D), lambda i: (i, 0)),
      ],
      out_shape=[jax.ShapeDtypeStruct((NRELS, NP, D), f32),
                 jax.ShapeDtypeStruct((NP, D), f32)],
  )(parts, selfb, wr, wl, b2)


def _tc_gat(parts, selfb, gat_w, a8):
  def wrapped(p0_ref, h_ref, gw_ref, a8_ref, o8_ref):
    h = jnp.maximum(p0_ref[0] + p0_ref[1] + h_ref[...], 0.0)
    z = jnp.dot(h, gw_ref[...], preferred_element_type=f32)
    o = jnp.dot(z, a8_ref[...], preferred_element_type=f32)
    o8_ref[...] = o.T

  return pl.pallas_call(
      wrapped,
      grid=(GRID,),
      in_specs=[
          pl.BlockSpec((2, BN, D), lambda i: (0, i, 0)),
          pl.BlockSpec((BN, D), lambda i: (i, 0)),
          pl.BlockSpec((D, D), lambda i: (0, 0)),
          pl.BlockSpec((D, 8), lambda i: (0, 0)),
      ],
      out_specs=pl.BlockSpec((8, BN), lambda i: (0, i)),
      out_shape=jax.ShapeDtypeStruct((8, NP), f32),
  )(parts, selfb, gat_w, a8)


def _tc_finish(den_parts, num_parts, gb2, dwt, db2):
  def wrapped(dp_ref, np_ref, gb_ref, dwt_ref, db_ref, o_ref):
    den = dp_ref[0] + dp_ref[1]
    num = np_ref[0] + np_ref[1]
    col = lax.broadcasted_iota(i32, (1, NP), 1)[0]
    valid = (col < N) & (den > 0)
    t = jnp.where(valid, num / jnp.where(den > 0, den, 1.0), 0.0)
    gbdw = jnp.sum(gb_ref[...] * dwt_ref[...])
    o_ref[...] = jnp.reshape(jnp.sum(t) + N * gbdw + db_ref[0, 0], (1, 1))

  return pl.pallas_call(
      wrapped,
      grid=(1,),
      in_specs=[
          pl.BlockSpec((2, NP), lambda i: (0, 0)),
          pl.BlockSpec((2, NP), lambda i: (0, 0)),
          pl.BlockSpec((1, D), lambda i: (0, 0)),
          pl.BlockSpec((1, D), lambda i: (0, 0)),
          pl.BlockSpec((1, 1), lambda i: (0, 0)),
      ],
      out_specs=pl.BlockSpec((1, 1), lambda i: (0, 0)),
      out_shape=jax.ShapeDtypeStruct((1, 1), f32),
  )(den_parts, num_parts, gb2, dwt, db2)


def kernel(x, W_rels, W_loops, b_rels, gat_W, gat_al, gat_ar, gat_b,
           dense_W, dense_b, edge_index, edge_type):
  src = edge_index[0].astype(i32)
  dst = edge_index[1].astype(i32)
  et = edge_type.astype(i32)

  npad = EP - E
  ar = jnp.arange(npad, dtype=i32)
  srcp = jnp.concatenate([src, (ar * 13) % N])
  typep = jnp.concatenate([et, ar % NRELS])
  dstp = jnp.concatenate([dst, N + (ar % (NP - N))])

  xp = jnp.concatenate([x, jnp.zeros((NP - N, D), f32)], axis=0)
  a8 = jnp.concatenate(
      [gat_al[:, None], gat_ar[:, None], dense_W,
       jnp.zeros((D, 5), f32)], axis=1)
  gb2 = gat_b[None, :]
  dwt = dense_W.T
  db2 = dense_b[None, :]

  hr, selfb = _tc_first(xp, W_rels[0], W_loops[0], b_rels[0][None, :])
  for l in range(1, NLAYERS + 1):
    parts = _conv_edges(hr.reshape(NRELS * NP, D), srcp, typep, dstp)
    if l < NLAYERS:
      hr, selfb = _tc_mid(parts, selfb, W_rels[l], W_loops[l],
                          b_rels[l][None, :])
  eo8 = _tc_gat(parts, selfb, gat_W, a8)
  m_parts = _gat_max(eo8, srcp, dstp)
  den_parts, num_parts = _gat_sums(eo8, srcp, dstp, m_parts)
  out = _tc_finish(den_parts, num_parts, gb2, dwt, db2)
  return out.reshape(1, 1, 1)


# speedup vs baseline: 49.9949x; 1.4362x over previous
import functools

import jax
import jax.numpy as jnp
from jax import lax
from jax.experimental import pallas as pl
from jax.experimental.pallas import tpu as pltpu
from jax.experimental.pallas import tpu_sc as plsc

N = 10000
D = 128
NRELS = 4
NLAYERS = 4
E = 320000

NP = 10240
EP = 327680
EW = EP // 32
CHUNK = 128
NCHUNKS = EW // CHUNK
ROWS_PER_SUB = NP // 16
NACC = 10112
ACC_PER_SUB = NACC // 16
BN = 1024
GRID = NP // BN

_mesh = functools.partial(
    plsc.VectorSubcoreMesh, core_axis_name="c", subcore_axis_name="s",
    num_cores=2, num_subcores=16)

_SC_PARAMS = pltpu.CompilerParams(needs_layout_passes=False)

f32 = jnp.float32
i32 = jnp.int32


def _conv_edges_body(hr, gidx3, dst3, out, gixb, dixb, rows3, acc,
                     g0, g1, g2, s0, s1, s2):
  gsem = [g0, g1, g2]
  ssem = [s0, s1, s2]
  c = lax.axis_index("c")
  s = lax.axis_index("s")
  wid = c * 16 + s

  @pl.loop(0, 128)
  def _(i):
    for j in range(D // 16):
      rows3[0, i, pl.ds(j * 16, 16)] = jnp.zeros((16,), f32)

  rowbase = s * ACC_PER_SUB
  for k in range(ACC_PER_SUB // 128):
    pltpu.sync_copy(rows3.at[0], acc.at[pl.ds(rowbase + k * 128, 128)])
  rem = ACC_PER_SUB % 128
  if rem:
    pltpu.sync_copy(rows3.at[0, pl.ds(0, rem)],
                    acc.at[pl.ds(rowbase + ACC_PER_SUB - rem, rem)])
  plsc.subcore_barrier()

  def load_gather_chunk(k, b):
    pltpu.sync_copy(gidx3.at[wid, k], gixb.at[b])
    pltpu.sync_copy(dst3.at[wid, k], dixb.at[b])
    pltpu.async_copy(hr.at[gixb.at[b]], rows3.at[b], gsem[b])

  def gather_wait(b):
    pltpu.make_async_copy(hr.at[gixb.at[b]], rows3.at[b], gsem[b]).wait()

  def scat(b):
    pltpu.async_copy(rows3.at[b], acc.at[dixb.at[b]], ssem[b], add=True)

  def scat_wait(b):
    pltpu.make_async_copy(rows3.at[b], acc.at[dixb.at[b]], ssem[b]).wait()

  load_gather_chunk(0, 0)

  def step(k, b, bp):
    @pl.when(k >= 2)
    def _():
      scat_wait(bp)

    @pl.when(k + 1 < NCHUNKS)
    def _():
      load_gather_chunk(k + 1, bp)

    gather_wait(b)
    scat(b)

  @pl.loop(0, (NCHUNKS - 2) // 3)
  def _(it):
    for bb in range(3):
      step(it * 3 + bb, bb, (bb + 1) % 3)

  step(NCHUNKS - 2, (NCHUNKS - 2) % 3, (NCHUNKS - 1) % 3)
  step(NCHUNKS - 1, (NCHUNKS - 1) % 3, NCHUNKS % 3)
  scat_wait((NCHUNKS - 2) % 3)
  scat_wait((NCHUNKS - 1) % 3)

  plsc.subcore_barrier()
  pltpu.sync_copy(acc.at[pl.ds(s * ACC_PER_SUB, ACC_PER_SUB)],
                  out.at[c, pl.ds(s * ACC_PER_SUB, ACC_PER_SUB)])


def _conv_edges(hr_flat, gidx3, dst3):
  return pl.kernel(
      _conv_edges_body,
      out_type=jax.ShapeDtypeStruct((2, NP, D), f32),
      mesh=_mesh(),
      compiler_params=_SC_PARAMS,
      scratch_types=[
          pltpu.VMEM((3, CHUNK), i32),
          pltpu.VMEM((3, CHUNK), i32),
          pltpu.VMEM((3, CHUNK, D), f32),
          pltpu.VMEM_SHARED((NACC, D), f32),
          pltpu.SemaphoreType.DMA,
          pltpu.SemaphoreType.DMA,
          pltpu.SemaphoreType.DMA,
          pltpu.SemaphoreType.DMA,
          pltpu.SemaphoreType.DMA,
          pltpu.SemaphoreType.DMA,
      ],
  )(hr_flat, gidx3, dst3)


def _gat_max_body(eo8, srcp, dstp, out, el_v, er_v, m_v, sv, dv,
                  redv, tmpv, msh):
  c = lax.axis_index("c")
  s = lax.axis_index("s")
  wid = c * 16 + s
  base = wid * EW

  pltpu.sync_copy(eo8.at[0], el_v)
  pltpu.sync_copy(eo8.at[1], er_v)
  pltpu.sync_copy(srcp.at[pl.ds(base, EW)], sv)
  pltpu.sync_copy(dstp.at[pl.ds(base, EW)], dv)

  @pl.loop(0, NP // 16)
  def _(i):
    m_v[pl.ds(i * 16, 16)] = jnp.full((16,), -jnp.inf, f32)

  @pl.loop(0, EW // 16)
  def _(k):
    sl = pl.ds(k * 16, 16)
    s16 = sv[sl]
    d16 = dv[sl]
    x = plsc.load_gather(el_v, [s16]) + plsc.load_gather(er_v, [d16])
    e16 = jnp.where(x > 0, x, 0.2 * x)
    cur = plsc.load_gather(m_v, [d16])

    def _again(cur):
      return jnp.max(jnp.where(e16 > cur, 1, 0)) > 0

    def _push(cur):
      plsc.store_scatter(m_v, [d16], jnp.maximum(e16, cur), mask=e16 > cur)
      return plsc.load_gather(m_v, [d16])

    lax.while_loop(_again, _push, cur)

  pltpu.sync_copy(m_v, msh.at[s])
  plsc.subcore_barrier()
  colbase = s * ROWS_PER_SUB
  pltpu.sync_copy(msh.at[0, pl.ds(colbase, ROWS_PER_SUB)], redv)
  for k in range(1, 16):
    pltpu.sync_copy(msh.at[k, pl.ds(colbase, ROWS_PER_SUB)], tmpv)
    for i in range(ROWS_PER_SUB // 16):
      sl = pl.ds(i * 16, 16)
      redv[sl] = jnp.maximum(redv[sl], tmpv[sl])
  pltpu.sync_copy(redv, out.at[c, pl.ds(colbase, ROWS_PER_SUB)])


def _gat_max(eo8, srcp, dstp):
  return pl.kernel(
      _gat_max_body,
      out_type=jax.ShapeDtypeStruct((2, NP), f32),
      mesh=_mesh(),
      compiler_params=_SC_PARAMS,
      scratch_types=[
          pltpu.VMEM((NP,), f32),
          pltpu.VMEM((NP,), f32),
          pltpu.VMEM((NP,), f32),
          pltpu.VMEM((EW,), i32),
          pltpu.VMEM((EW,), i32),
          pltpu.VMEM((ROWS_PER_SUB,), f32),
          pltpu.VMEM((ROWS_PER_SUB,), f32),
          pltpu.VMEM_SHARED((16, NP), f32),
      ],
  )(eo8, srcp, dstp)


def _gat_sums_body(eo8, srcp, dstp, m_parts, den_out, num_out,
                   el_v, er_v, p_v, m_v, mtmp, sbuf, dbuf,
                   eebuf, epbuf, didx2, redv, den_sp, num_sp):
  c = lax.axis_index("c")
  s = lax.axis_index("s")
  wid = c * 16 + s
  base = wid * EW

  pltpu.sync_copy(eo8.at[0], el_v)
  pltpu.sync_copy(eo8.at[1], er_v)
  pltpu.sync_copy(eo8.at[2], p_v)
  pltpu.sync_copy(m_parts.at[0], m_v)
  pltpu.sync_copy(m_parts.at[1], mtmp)
  pltpu.sync_copy(srcp.at[pl.ds(base, EW)], sbuf)
  pltpu.sync_copy(dstp.at[pl.ds(base, EW)], dbuf)

  @pl.loop(0, NP // 16)
  def _(i):
    sl = pl.ds(i * 16, 16)
    m_v[sl] = jnp.maximum(m_v[sl], mtmp[sl])

  @pl.loop(0, ROWS_PER_SUB // 16)
  def _(i):
    redv[pl.ds(i * 16, 16)] = jnp.zeros((16,), f32)
  colbase = s * ROWS_PER_SUB
  pltpu.sync_copy(redv, den_sp.at[pl.ds(colbase, ROWS_PER_SUB)])
  pltpu.sync_copy(redv, num_sp.at[pl.ds(colbase, ROWS_PER_SUB)])
  plsc.subcore_barrier()

  @pl.loop(0, EW // 512)
  def _(k):
    off = k * 512
    for j in range(32):
      sl = pl.ds(off + j * 16, 16)
      s16 = sbuf[sl]
      d16 = dbuf[sl]
      x = plsc.load_gather(el_v, [s16]) + plsc.load_gather(er_v, [d16])
      e16 = jnp.where(x > 0, x, 0.2 * x)
      ee = jnp.exp(e16 - plsc.load_gather(m_v, [d16]))
      pg = plsc.load_gather(p_v, [s16])
      r, col = j // 8, pl.ds((j % 8) * 16, 16)
      eebuf[r, col] = ee
      epbuf[r, col] = ee * pg
      didx2[r, col] = d16
    for r in range(4):
      pltpu.sync_copy(eebuf.at[r], den_sp.at[didx2.at[r]], add=True)
      pltpu.sync_copy(epbuf.at[r], num_sp.at[didx2.at[r]], add=True)

  plsc.subcore_barrier()
  pltpu.sync_copy(den_sp.at[pl.ds(colbase, ROWS_PER_SUB)], redv)
  pltpu.sync_copy(redv, den_out.at[c, pl.ds(colbase, ROWS_PER_SUB)])
  pltpu.sync_copy(num_sp.at[pl.ds(colbase, ROWS_PER_SUB)], redv)
  pltpu.sync_copy(redv, num_out.at[c, pl.ds(colbase, ROWS_PER_SUB)])


def _gat_sums(eo8, srcp, dstp, m_parts):
  return pl.kernel(
      _gat_sums_body,
      out_type=[jax.ShapeDtypeStruct((2, NP), f32),
                jax.ShapeDtypeStruct((2, NP), f32)],
      mesh=_mesh(),
      compiler_params=_SC_PARAMS,
      scratch_types=[
          pltpu.VMEM((NP,), f32),
          pltpu.VMEM((NP,), f32),
          pltpu.VMEM((NP,), f32),
          pltpu.VMEM((NP,), f32),
          pltpu.VMEM((NP,), f32),
          pltpu.VMEM((EW,), i32),
          pltpu.VMEM((EW,), i32),
          pltpu.VMEM((4, 128), f32),
          pltpu.VMEM((4, 128), f32),
          pltpu.VMEM((4, 128), i32),
          pltpu.VMEM((ROWS_PER_SUB,), f32),
          pltpu.VMEM_SHARED((NP,), f32),
          pltpu.VMEM_SHARED((NP,), f32),
      ],
  )(eo8, srcp, dstp, m_parts)


def _mm_step_body(relu_in, h_ref, wr_ref, wl_ref, b_ref, hr_ref, selfb_ref,
                  p0_ref=None):
  if relu_in:
    h = jnp.maximum(p0_ref[0] + p0_ref[1] + h_ref[...], 0.0)
  else:
    h = h_ref[...]
  for r in range(NRELS):
    hr_ref[r] = jnp.dot(h, wr_ref[r], preferred_element_type=f32)
  selfb_ref[...] = jnp.dot(h, wl_ref[...], preferred_element_type=f32) + b_ref[...]


def _tc_first(xp, wr, wl, b2):
  body = functools.partial(_mm_step_body, False)

  def wrapped(h_ref, wr_ref, wl_ref, b_ref, hr_ref, selfb_ref):
    body(h_ref, wr_ref, wl_ref, b_ref, hr_ref, selfb_ref)

  return pl.pallas_call(
      wrapped,
      grid=(GRID,),
      in_specs=[
          pl.BlockSpec((BN, D), lambda i: (i, 0)),
          pl.BlockSpec((NRELS, D, D), lambda i: (0, 0, 0)),
          pl.BlockSpec((D, D), lambda i: (0, 0)),
          pl.BlockSpec((1, D), lambda i: (0, 0)),
      ],
      out_specs=[
          pl.BlockSpec((NRELS, BN, D), lambda i: (0, i, 0)),
          pl.BlockSpec((BN, D), lambda i: (i, 0)),
      ],
      out_shape=[jax.ShapeDtypeStruct((NRELS, NP, D), f32),
                 jax.ShapeDtypeStruct((NP, D), f32)],
  )(xp, wr, wl, b2)


def _tc_mid(parts, selfb, wr, wl, b2):
  def wrapped(p0_ref, h_ref, wr_ref, wl_ref, b_ref, hr_ref, selfb_ref):
    _mm_step_body(True, h_ref, wr_ref, wl_ref, b_ref, hr_ref, selfb_ref,
                  p0_ref=p0_ref)

  return pl.pallas_call(
      wrapped,
      grid=(GRID,),
      in_specs=[
          pl.BlockSpec((2, BN, D), lambda i: (0, i, 0)),
          pl.BlockSpec((BN, D), lambda i: (i, 0)),
          pl.BlockSpec((NRELS, D, D), lambda i: (0, 0, 0)),
          pl.BlockSpec((D, D), lambda i: (0, 0)),
          pl.BlockSpec((1, D), lambda i: (0, 0)),
      ],
      out_specs=[
          pl.BlockSpec((NRELS, BN, D), lambda i: (0, i, 0)),
          pl.BlockSpec((BN, D), lambda i: (i, 0)),
      ],
      out_shape=[jax.ShapeDtypeStruct((NRELS, NP, D), f32),
                 jax.ShapeDtypeStruct((NP, D), f32)],
  )(parts, selfb, wr, wl, b2)


def _tc_gat(parts, selfb, gat_w, a8):
  def wrapped(p0_ref, h_ref, gw_ref, a8_ref, o8_ref):
    h = jnp.maximum(p0_ref[0] + p0_ref[1] + h_ref[...], 0.0)
    z = jnp.dot(h, gw_ref[...], preferred_element_type=f32)
    o = jnp.dot(z, a8_ref[...], preferred_element_type=f32)
    o8_ref[...] = o.T

  return pl.pallas_call(
      wrapped,
      grid=(GRID,),
      in_specs=[
          pl.BlockSpec((2, BN, D), lambda i: (0, i, 0)),
          pl.BlockSpec((BN, D), lambda i: (i, 0)),
          pl.BlockSpec((D, D), lambda i: (0, 0)),
          pl.BlockSpec((D, 8), lambda i: (0, 0)),
      ],
      out_specs=pl.BlockSpec((8, BN), lambda i: (0, i)),
      out_shape=jax.ShapeDtypeStruct((8, NP), f32),
  )(parts, selfb, gat_w, a8)


def _tc_finish(den_parts, num_parts, gb2, dwt, db2):
  def wrapped(dp_ref, np_ref, gb_ref, dwt_ref, db_ref, o_ref):
    den = dp_ref[0] + dp_ref[1]
    num = np_ref[0] + np_ref[1]
    col = lax.broadcasted_iota(i32, (1, NP), 1)[0]
    valid = (col < N) & (den > 0)
    t = jnp.where(valid, num / jnp.where(den > 0, den, 1.0), 0.0)
    gbdw = jnp.sum(gb_ref[...] * dwt_ref[...])
    o_ref[...] = jnp.reshape(jnp.sum(t) + N * gbdw + db_ref[0, 0], (1, 1))

  return pl.pallas_call(
      wrapped,
      grid=(1,),
      in_specs=[
          pl.BlockSpec((2, NP), lambda i: (0, 0)),
          pl.BlockSpec((2, NP), lambda i: (0, 0)),
          pl.BlockSpec((1, D), lambda i: (0, 0)),
          pl.BlockSpec((1, D), lambda i: (0, 0)),
          pl.BlockSpec((1, 1), lambda i: (0, 0)),
      ],
      out_specs=pl.BlockSpec((1, 1), lambda i: (0, 0)),
      out_shape=jax.ShapeDtypeStruct((1, 1), f32),
  )(den_parts, num_parts, gb2, dwt, db2)


def kernel(x, W_rels, W_loops, b_rels, gat_W, gat_al, gat_ar, gat_b,
           dense_W, dense_b, edge_index, edge_type):
  src = edge_index[0].astype(i32)
  dst = edge_index[1].astype(i32)
  et = edge_type.astype(i32)

  npad = EP - E
  ar = jnp.arange(npad, dtype=i32)
  srcp = jnp.concatenate([src, (ar * 13) % N])
  typep = jnp.concatenate([et, ar % NRELS])
  dstp = jnp.concatenate([dst, N + (ar % (NACC - N))])
  gidx3 = (typep * NP + srcp).reshape(32, NCHUNKS, CHUNK)
  dst3 = dstp.reshape(32, NCHUNKS, CHUNK)

  xp = jnp.concatenate([x, jnp.zeros((NP - N, D), f32)], axis=0)
  a8 = jnp.concatenate(
      [gat_al[:, None], gat_ar[:, None], dense_W,
       jnp.zeros((D, 5), f32)], axis=1)
  gb2 = gat_b[None, :]
  dwt = dense_W.T
  db2 = dense_b[None, :]

  hr, selfb = _tc_first(xp, W_rels[0], W_loops[0], b_rels[0][None, :])
  for l in range(1, NLAYERS + 1):
    parts = _conv_edges(hr.reshape(NRELS * NP, D), gidx3, dst3)
    if l < NLAYERS:
      hr, selfb = _tc_mid(parts, selfb, W_rels[l], W_loops[l],
                          b_rels[l][None, :])
  eo8 = _tc_gat(parts, selfb, gat_W, a8)
  m_parts = _gat_max(eo8, srcp, dstp)
  den_parts, num_parts = _gat_sums(eo8, srcp, dstp, m_parts)
  out = _tc_finish(den_parts, num_parts, gb2, dwt, db2)
  return out.reshape(1, 1, 1)


# speedup vs baseline: 51.1348x; 1.0228x over previous
import functools

import jax
import jax.numpy as jnp
from jax import lax
from jax.experimental import pallas as pl
from jax.experimental.pallas import tpu as pltpu
from jax.experimental.pallas import tpu_sc as plsc

N = 10000
D = 128
NRELS = 4
NLAYERS = 4
E = 320000

NP = 10240
EP = 327680
EW = EP // 32
CHUNK = 128
NCHUNKS = EW // CHUNK
ROWS_PER_SUB = NP // 16
NACC = 10112
ACC_PER_SUB = NACC // 16
BN = 1024
GRID = NP // BN

_mesh = functools.partial(
    plsc.VectorSubcoreMesh, core_axis_name="c", subcore_axis_name="s",
    num_cores=2, num_subcores=16)

_SC_PARAMS = pltpu.CompilerParams(needs_layout_passes=False)

f32 = jnp.float32
i32 = jnp.int32


def _conv_edges_body(hr, gidx3, dst3, out, gixb, dixb, rows3, acc,
                     g0, g1, g2, s0, s1, s2):
  gsem = [g0, g1, g2]
  ssem = [s0, s1, s2]
  c = lax.axis_index("c")
  s = lax.axis_index("s")
  wid = c * 16 + s

  @pl.loop(0, 128)
  def _(i):
    for j in range(D // 16):
      rows3[0, i, pl.ds(j * 16, 16)] = jnp.zeros((16,), f32)

  rowbase = s * ACC_PER_SUB
  for k in range(ACC_PER_SUB // 128):
    pltpu.sync_copy(rows3.at[0], acc.at[pl.ds(rowbase + k * 128, 128)])
  rem = ACC_PER_SUB % 128
  if rem:
    pltpu.sync_copy(rows3.at[0, pl.ds(0, rem)],
                    acc.at[pl.ds(rowbase + ACC_PER_SUB - rem, rem)])
  plsc.subcore_barrier()

  def load_gather_chunk(k, b):
    pltpu.sync_copy(gidx3.at[wid, k], gixb.at[b])
    pltpu.sync_copy(dst3.at[wid, k], dixb.at[b])
    pltpu.async_copy(hr.at[gixb.at[b]], rows3.at[b], gsem[b])

  def gather_wait(b):
    pltpu.make_async_copy(hr.at[gixb.at[b]], rows3.at[b], gsem[b]).wait()

  def scat(b):
    pltpu.async_copy(rows3.at[b], acc.at[dixb.at[b]], ssem[b], add=True)

  def scat_wait(b):
    pltpu.make_async_copy(rows3.at[b], acc.at[dixb.at[b]], ssem[b]).wait()

  load_gather_chunk(0, 0)

  def step(k, b, bp):
    @pl.when(k >= 2)
    def _():
      scat_wait(bp)

    @pl.when(k + 1 < NCHUNKS)
    def _():
      load_gather_chunk(k + 1, bp)

    gather_wait(b)
    scat(b)

  @pl.loop(0, (NCHUNKS - 2) // 3)
  def _(it):
    for bb in range(3):
      step(it * 3 + bb, bb, (bb + 1) % 3)

  step(NCHUNKS - 2, (NCHUNKS - 2) % 3, (NCHUNKS - 1) % 3)
  step(NCHUNKS - 1, (NCHUNKS - 1) % 3, NCHUNKS % 3)
  scat_wait((NCHUNKS - 2) % 3)
  scat_wait((NCHUNKS - 1) % 3)

  plsc.subcore_barrier()
  pltpu.sync_copy(acc.at[pl.ds(s * ACC_PER_SUB, ACC_PER_SUB)],
                  out.at[c, pl.ds(s * ACC_PER_SUB, ACC_PER_SUB)])


def _conv_edges(hr_flat, gidx3, dst3):
  return pl.kernel(
      _conv_edges_body,
      out_type=jax.ShapeDtypeStruct((2, NP, D), f32),
      mesh=_mesh(),
      compiler_params=_SC_PARAMS,
      scratch_types=[
          pltpu.VMEM((3, CHUNK), i32),
          pltpu.VMEM((3, CHUNK), i32),
          pltpu.VMEM((3, CHUNK, D), f32),
          pltpu.VMEM_SHARED((NACC, D), f32),
          pltpu.SemaphoreType.DMA,
          pltpu.SemaphoreType.DMA,
          pltpu.SemaphoreType.DMA,
          pltpu.SemaphoreType.DMA,
          pltpu.SemaphoreType.DMA,
          pltpu.SemaphoreType.DMA,
      ],
  )(hr_flat, gidx3, dst3)


def _gat_max_body(eo8, srcp, dstp, out, el_v, er_v, m_v, sv, dv,
                  redv, tmpv, msh):
  c = lax.axis_index("c")
  s = lax.axis_index("s")
  wid = c * 16 + s
  base = wid * EW

  pltpu.sync_copy(eo8.at[0], el_v)
  pltpu.sync_copy(eo8.at[1], er_v)
  pltpu.sync_copy(srcp.at[pl.ds(base, EW)], sv)
  pltpu.sync_copy(dstp.at[pl.ds(base, EW)], dv)

  @pl.loop(0, NP // 16)
  def _(i):
    m_v[pl.ds(i * 16, 16)] = jnp.full((16,), -jnp.inf, f32)

  @pl.loop(0, EW // 16, unroll=4)
  def _(k):
    sl = pl.ds(k * 16, 16)
    s16 = sv[sl]
    d16 = dv[sl]
    x = plsc.load_gather(el_v, [s16]) + plsc.load_gather(er_v, [d16])
    e16 = jnp.where(x > 0, x, 0.2 * x)
    cur = plsc.load_gather(m_v, [d16])

    def _again(cur):
      return jnp.max(jnp.where(e16 > cur, 1, 0)) > 0

    def _push(cur):
      plsc.store_scatter(m_v, [d16], jnp.maximum(e16, cur), mask=e16 > cur)
      return plsc.load_gather(m_v, [d16])

    lax.while_loop(_again, _push, cur)

  pltpu.sync_copy(m_v, msh.at[s])
  plsc.subcore_barrier()
  colbase = s * ROWS_PER_SUB
  pltpu.sync_copy(msh.at[0, pl.ds(colbase, ROWS_PER_SUB)], redv)
  for k in range(1, 16):
    pltpu.sync_copy(msh.at[k, pl.ds(colbase, ROWS_PER_SUB)], tmpv)
    for i in range(ROWS_PER_SUB // 16):
      sl = pl.ds(i * 16, 16)
      redv[sl] = jnp.maximum(redv[sl], tmpv[sl])
  pltpu.sync_copy(redv, out.at[c, pl.ds(colbase, ROWS_PER_SUB)])


def _gat_max(eo8, srcp, dstp):
  return pl.kernel(
      _gat_max_body,
      out_type=jax.ShapeDtypeStruct((2, NP), f32),
      mesh=_mesh(),
      compiler_params=_SC_PARAMS,
      scratch_types=[
          pltpu.VMEM((NP,), f32),
          pltpu.VMEM((NP,), f32),
          pltpu.VMEM((NP,), f32),
          pltpu.VMEM((EW,), i32),
          pltpu.VMEM((EW,), i32),
          pltpu.VMEM((ROWS_PER_SUB,), f32),
          pltpu.VMEM((ROWS_PER_SUB,), f32),
          pltpu.VMEM_SHARED((16, NP), f32),
      ],
  )(eo8, srcp, dstp)


def _gat_sums_body(eo8, srcp, dstp, m_parts, den_out, num_out,
                   el_v, er_v, p_v, m_v, mtmp, sbuf, dbuf,
                   eebuf, epbuf, didx2, redv, den_sp, num_sp, q0, q1):
  qsem = [q0, q1]
  c = lax.axis_index("c")
  s = lax.axis_index("s")
  wid = c * 16 + s
  base = wid * EW

  pltpu.sync_copy(eo8.at[0], el_v)
  pltpu.sync_copy(eo8.at[1], er_v)
  pltpu.sync_copy(eo8.at[2], p_v)
  pltpu.sync_copy(m_parts.at[0], m_v)
  pltpu.sync_copy(m_parts.at[1], mtmp)
  pltpu.sync_copy(srcp.at[pl.ds(base, EW)], sbuf)
  pltpu.sync_copy(dstp.at[pl.ds(base, EW)], dbuf)

  @pl.loop(0, NP // 16)
  def _(i):
    sl = pl.ds(i * 16, 16)
    m_v[sl] = jnp.maximum(m_v[sl], mtmp[sl])

  @pl.loop(0, ROWS_PER_SUB // 16)
  def _(i):
    redv[pl.ds(i * 16, 16)] = jnp.zeros((16,), f32)
  colbase = s * ROWS_PER_SUB
  pltpu.sync_copy(redv, den_sp.at[pl.ds(colbase, ROWS_PER_SUB)])
  pltpu.sync_copy(redv, num_sp.at[pl.ds(colbase, ROWS_PER_SUB)])
  plsc.subcore_barrier()

  def drain(hb):
    for r in range(4):
      pltpu.make_async_copy(eebuf.at[hb, r], den_sp.at[didx2.at[hb, r]],
                            qsem[hb]).wait()
      pltpu.make_async_copy(epbuf.at[hb, r], num_sp.at[didx2.at[hb, r]],
                            qsem[hb]).wait()

  @pl.loop(0, EW // 1024)
  def _(k):
    for hb in range(2):
      kk = k * 2 + hb
      off = kk * 512

      @pl.when(kk >= 2)
      def _():
        drain(hb)

      for j in range(32):
        sl = pl.ds(off + j * 16, 16)
        s16 = sbuf[sl]
        d16 = dbuf[sl]
        x = plsc.load_gather(el_v, [s16]) + plsc.load_gather(er_v, [d16])
        e16 = jnp.where(x > 0, x, 0.2 * x)
        ee = jnp.exp(e16 - plsc.load_gather(m_v, [d16]))
        pg = plsc.load_gather(p_v, [s16])
        r, col = j // 8, pl.ds((j % 8) * 16, 16)
        eebuf[hb, r, col] = ee
        epbuf[hb, r, col] = ee * pg
        didx2[hb, r, col] = d16
      for r in range(4):
        pltpu.async_copy(eebuf.at[hb, r], den_sp.at[didx2.at[hb, r]],
                         qsem[hb], add=True)
        pltpu.async_copy(epbuf.at[hb, r], num_sp.at[didx2.at[hb, r]],
                         qsem[hb], add=True)

  drain(0)
  drain(1)
  plsc.subcore_barrier()
  pltpu.sync_copy(den_sp.at[pl.ds(colbase, ROWS_PER_SUB)], redv)
  pltpu.sync_copy(redv, den_out.at[c, pl.ds(colbase, ROWS_PER_SUB)])
  pltpu.sync_copy(num_sp.at[pl.ds(colbase, ROWS_PER_SUB)], redv)
  pltpu.sync_copy(redv, num_out.at[c, pl.ds(colbase, ROWS_PER_SUB)])


def _gat_sums(eo8, srcp, dstp, m_parts):
  return pl.kernel(
      _gat_sums_body,
      out_type=[jax.ShapeDtypeStruct((2, NP), f32),
                jax.ShapeDtypeStruct((2, NP), f32)],
      mesh=_mesh(),
      compiler_params=_SC_PARAMS,
      scratch_types=[
          pltpu.VMEM((NP,), f32),
          pltpu.VMEM((NP,), f32),
          pltpu.VMEM((NP,), f32),
          pltpu.VMEM((NP,), f32),
          pltpu.VMEM((NP,), f32),
          pltpu.VMEM((EW,), i32),
          pltpu.VMEM((EW,), i32),
          pltpu.VMEM((2, 4, 128), f32),
          pltpu.VMEM((2, 4, 128), f32),
          pltpu.VMEM((2, 4, 128), i32),
          pltpu.VMEM((ROWS_PER_SUB,), f32),
          pltpu.VMEM_SHARED((NP,), f32),
          pltpu.VMEM_SHARED((NP,), f32),
          pltpu.SemaphoreType.DMA,
          pltpu.SemaphoreType.DMA,
      ],
  )(eo8, srcp, dstp, m_parts)


def _mm_step_body(relu_in, h_ref, wr_ref, wl_ref, b_ref, hr_ref, selfb_ref,
                  p0_ref=None):
  if relu_in:
    h = jnp.maximum(p0_ref[0] + p0_ref[1] + h_ref[...], 0.0)
  else:
    h = h_ref[...]
  for r in range(NRELS):
    hr_ref[r] = jnp.dot(h, wr_ref[r], preferred_element_type=f32)
  selfb_ref[...] = jnp.dot(h, wl_ref[...], preferred_element_type=f32) + b_ref[...]


def _tc_first(xp, wr, wl, b2):
  body = functools.partial(_mm_step_body, False)

  def wrapped(h_ref, wr_ref, wl_ref, b_ref, hr_ref, selfb_ref):
    body(h_ref, wr_ref, wl_ref, b_ref, hr_ref, selfb_ref)

  return pl.pallas_call(
      wrapped,
      grid=(GRID,),
      in_specs=[
          pl.BlockSpec((BN, D), lambda i: (i, 0)),
          pl.BlockSpec((NRELS, D, D), lambda i: (0, 0, 0)),
          pl.BlockSpec((D, D), lambda i: (0, 0)),
          pl.BlockSpec((1, D), lambda i: (0, 0)),
      ],
      out_specs=[
          pl.BlockSpec((NRELS, BN, D), lambda i: (0, i, 0)),
          pl.BlockSpec((BN, D), lambda i: (i, 0)),
      ],
      out_shape=[jax.ShapeDtypeStruct((NRELS, NP, D), f32),
                 jax.ShapeDtypeStruct((NP, D), f32)],
  )(xp, wr, wl, b2)


def _tc_mid(parts, selfb, wr, wl, b2):
  def wrapped(p0_ref, h_ref, wr_ref, wl_ref, b_ref, hr_ref, selfb_ref):
    _mm_step_body(True, h_ref, wr_ref, wl_ref, b_ref, hr_ref, selfb_ref,
                  p0_ref=p0_ref)

  return pl.pallas_call(
      wrapped,
      grid=(GRID,),
      in_specs=[
          pl.BlockSpec((2, BN, D), lambda i: (0, i, 0)),
          pl.BlockSpec((BN, D), lambda i: (i, 0)),
          pl.BlockSpec((NRELS, D, D), lambda i: (0, 0, 0)),
          pl.BlockSpec((D, D), lambda i: (0, 0)),
          pl.BlockSpec((1, D), lambda i: (0, 0)),
      ],
      out_specs=[
          pl.BlockSpec((NRELS, BN, D), lambda i: (0, i, 0)),
          pl.BlockSpec((BN, D), lambda i: (i, 0)),
      ],
      out_shape=[jax.ShapeDtypeStruct((NRELS, NP, D), f32),
                 jax.ShapeDtypeStruct((NP, D), f32)],
  )(parts, selfb, wr, wl, b2)


def _tc_gat(parts, selfb, gat_w, a8):
  def wrapped(p0_ref, h_ref, gw_ref, a8_ref, o8_ref):
    h = jnp.maximum(p0_ref[0] + p0_ref[1] + h_ref[...], 0.0)
    z = jnp.dot(h, gw_ref[...], preferred_element_type=f32)
    o = jnp.dot(z, a8_ref[...], preferred_element_type=f32)
    o8_ref[...] = o.T

  return pl.pallas_call(
      wrapped,
      grid=(GRID,),
      in_specs=[
          pl.BlockSpec((2, BN, D), lambda i: (0, i, 0)),
          pl.BlockSpec((BN, D), lambda i: (i, 0)),
          pl.BlockSpec((D, D), lambda i: (0, 0)),
          pl.BlockSpec((D, 8), lambda i: (0, 0)),
      ],
      out_specs=pl.BlockSpec((8, BN), lambda i: (0, i)),
      out_shape=jax.ShapeDtypeStruct((8, NP), f32),
  )(parts, selfb, gat_w, a8)


def _tc_finish(den_parts, num_parts, gb2, dwt, db2):
  def wrapped(dp_ref, np_ref, gb_ref, dwt_ref, db_ref, o_ref):
    den = dp_ref[0] + dp_ref[1]
    num = np_ref[0] + np_ref[1]
    col = lax.broadcasted_iota(i32, (1, NP), 1)[0]
    valid = (col < N) & (den > 0)
    t = jnp.where(valid, num / jnp.where(den > 0, den, 1.0), 0.0)
    gbdw = jnp.sum(gb_ref[...] * dwt_ref[...])
    o_ref[...] = jnp.reshape(jnp.sum(t) + N * gbdw + db_ref[0, 0], (1, 1))

  return pl.pallas_call(
      wrapped,
      grid=(1,),
      in_specs=[
          pl.BlockSpec((2, NP), lambda i: (0, 0)),
          pl.BlockSpec((2, NP), lambda i: (0, 0)),
          pl.BlockSpec((1, D), lambda i: (0, 0)),
          pl.BlockSpec((1, D), lambda i: (0, 0)),
          pl.BlockSpec((1, 1), lambda i: (0, 0)),
      ],
      out_specs=pl.BlockSpec((1, 1), lambda i: (0, 0)),
      out_shape=jax.ShapeDtypeStruct((1, 1), f32),
  )(den_parts, num_parts, gb2, dwt, db2)


def kernel(x, W_rels, W_loops, b_rels, gat_W, gat_al, gat_ar, gat_b,
           dense_W, dense_b, edge_index, edge_type):
  src = edge_index[0].astype(i32)
  dst = edge_index[1].astype(i32)
  et = edge_type.astype(i32)

  npad = EP - E
  ar = jnp.arange(npad, dtype=i32)
  srcp = jnp.concatenate([src, (ar * 13) % N])
  typep = jnp.concatenate([et, ar % NRELS])
  dstp = jnp.concatenate([dst, N + (ar % (NACC - N))])
  gidx3 = (typep * NP + srcp).reshape(32, NCHUNKS, CHUNK)
  dst3 = dstp.reshape(32, NCHUNKS, CHUNK)

  xp = jnp.concatenate([x, jnp.zeros((NP - N, D), f32)], axis=0)
  a8 = jnp.concatenate(
      [gat_al[:, None], gat_ar[:, None], dense_W,
       jnp.zeros((D, 5), f32)], axis=1)
  gb2 = gat_b[None, :]
  dwt = dense_W.T
  db2 = dense_b[None, :]

  hr, selfb = _tc_first(xp, W_rels[0], W_loops[0], b_rels[0][None, :])
  for l in range(1, NLAYERS + 1):
    parts = _conv_edges(hr.reshape(NRELS * NP, D), gidx3, dst3)
    if l < NLAYERS:
      hr, selfb = _tc_mid(parts, selfb, W_rels[l], W_loops[l],
                          b_rels[l][None, :])
  eo8 = _tc_gat(parts, selfb, gat_W, a8)
  m_parts = _gat_max(eo8, srcp, dstp)
  den_parts, num_parts = _gat_sums(eo8, srcp, dstp, m_parts)
  out = _tc_finish(den_parts, num_parts, gb2, dwt, db2)
  return out.reshape(1, 1, 1)
